# Initial kernel scaffold; baseline (speedup 1.0000x reference)
import jax
import jax.numpy as jnp
from jax.experimental import pallas as pl


def kernel(inputs, edge_index, W0, al0, ar0, W1, al1, ar1, W2, al2, ar2):
    raise NotImplementedError("write your pallas kernel here")


# speedup vs baseline: 23.9600x; 23.9600x over previous
import functools

import jax
import jax.numpy as jnp
from jax import lax
from jax.experimental import pallas as pl
from jax.experimental.pallas import tpu as pltpu
from jax.experimental.pallas import tpu_sc as plsc

N = 10000
NP = 10240
E = 320000
M = 128
LANES = 16
NC = 2
NS = 16
NW = NC * NS
EW = E // NW
CHUNK = 80
NCH = EW // CHUNK
ROWS_T = NP // NS
ZR = 128

_f32 = jnp.float32


def _dense_body(x_ref, w_ref, a_ref, feat_ref, t_ref, *, first):
    x = x_ref[...]
    if not first:
        t = x[0] + x[1]
        x = jnp.where(t > 0, t, jnp.exp(t) - 1.0)
    feat = jnp.dot(x, w_ref[...], preferred_element_type=_f32)
    feat_ref[...] = feat
    t_ref[...] = jnp.dot(feat, a_ref[...], preferred_element_type=_f32)


def _dense_stage(x, w, a_comb, *, first):
    bn = 512
    grid = (NP // bn,)
    if first:
        x_spec = pl.BlockSpec((bn, M), lambda i: (i, 0))
    else:
        x_spec = pl.BlockSpec((2, bn, M), lambda i: (0, i, 0))
    return pl.pallas_call(
        functools.partial(_dense_body, first=first),
        grid=grid,
        in_specs=[
            x_spec,
            pl.BlockSpec((M, M), lambda i: (0, 0)),
            pl.BlockSpec((M, M), lambda i: (0, 0)),
        ],
        out_specs=[
            pl.BlockSpec((bn, M), lambda i: (i, 0)),
            pl.BlockSpec((bn, M), lambda i: (i, 0)),
        ],
        out_shape=[
            jax.ShapeDtypeStruct((NP, M), _f32),
            jax.ShapeDtypeStruct((NP, M), _f32),
        ],
    )(x, w, a_comb)


def _recip_body(s_ref, rs_ref):
    rs_ref[...] = 1.0 / (s_ref[0] + s_ref[1] + 1e-9)


def _recip_stage(s_part):
    bn = 512
    return pl.pallas_call(
        _recip_body,
        grid=(NP // bn,),
        in_specs=[pl.BlockSpec((2, bn, M), lambda i: (0, i, 0))],
        out_specs=pl.BlockSpec((bn, M), lambda i: (i, 0)),
        out_shape=jax.ShapeDtypeStruct((NP, M), _f32),
    )(s_part)


def _combine_body(p_ref, o_ref):
    o_ref[...] = p_ref[0] + p_ref[1]


def _combine_stage(p):
    bn = 512
    return pl.pallas_call(
        _combine_body,
        grid=(NP // bn,),
        in_specs=[pl.BlockSpec((2, bn, M), lambda i: (0, i, 0))],
        out_specs=pl.BlockSpec((bn, M), lambda i: (i, 0)),
        out_shape=jax.ShapeDtypeStruct((NP, M), _f32),
    )(p)


def _zero_shared_slice(zrow, shared, sid):
    def zfill(i, _):
        r = i // (M // LANES)
        g = i % (M // LANES)
        zrow[r, pl.ds(g * LANES, LANES)] = jnp.zeros((LANES,), _f32)
        return 0

    lax.fori_loop(0, ZR * (M // LANES), zfill, 0)
    row0 = sid * ROWS_T
    for j in range(ROWS_T // ZR):
        pltpu.sync_copy(zrow, shared.at[pl.ds(row0 + j * ZR, ZR)])


def _sc_logits_body(t_hbm, src_hbm, dst_hbm, ee_hbm, spart_hbm,
                    sidx, didx, ts_v, td_v, ee1, eew, zrow, sacc,
                    *, heads):
    cid = lax.axis_index("c")
    sid = lax.axis_index("s")
    base0 = (cid * NS + sid) * EW

    _zero_shared_slice(zrow, sacc, sid)
    plsc.subcore_barrier()

    lane = lax.broadcasted_iota(jnp.int32, (LANES,), 0)

    def chunk_loop(c, _):
        eb = base0 + c * CHUNK
        pltpu.sync_copy(src_hbm.at[pl.ds(eb, CHUNK)], sidx)
        pltpu.sync_copy(dst_hbm.at[pl.ds(eb, CHUNK)], didx)
        pltpu.sync_copy(t_hbm.at[sidx], ts_v)
        pltpu.sync_copy(t_hbm.at[didx], td_v)

        def body(e, _):
            x = ts_v[e, pl.ds(0, LANES)] + td_v[e, pl.ds(LANES, LANES)]
            y = jnp.maximum(x, 0.2 * x)
            z = jnp.where(lane < heads, jnp.exp(y), 0.0)
            ee1[pl.ds(e * LANES, LANES)] = z
            if heads == 8:
                for g in range(8):
                    sp = jnp.full((LANES,), z[g], _f32)
                    eew[e, pl.ds(g * LANES, LANES)] = sp
            else:
                sp = jnp.full((LANES,), z[0], _f32)
                for g in range(8):
                    eew[e, pl.ds(g * LANES, LANES)] = sp
            return 0

        lax.fori_loop(0, CHUNK, body, 0)
        pltpu.sync_copy(ee1, ee_hbm.at[pl.ds(eb * LANES, CHUNK * LANES)])
        pltpu.sync_copy(eew, sacc.at[didx], add=True)
        return 0

    lax.fori_loop(0, NCH, chunk_loop, 0)

    plsc.subcore_barrier()
    row0 = sid * ROWS_T
    pltpu.sync_copy(sacc.at[pl.ds(row0, ROWS_T)],
                    spart_hbm.at[cid, pl.ds(row0, ROWS_T)])


def _sc_logits_stage(t_comb, src, dst, *, heads):
    mesh = plsc.VectorSubcoreMesh(core_axis_name="c", subcore_axis_name="s",
                                  num_cores=NC, num_subcores=NS)
    return pl.kernel(
        functools.partial(_sc_logits_body, heads=heads),
        out_type=(
            jax.ShapeDtypeStruct((E * LANES,), _f32),
            jax.ShapeDtypeStruct((NC, NP, M), _f32),
        ),
        mesh=mesh,
        scratch_types=[
            pltpu.VMEM((CHUNK,), jnp.int32),
            pltpu.VMEM((CHUNK,), jnp.int32),
            pltpu.VMEM((CHUNK, M), _f32),
            pltpu.VMEM((CHUNK, M), _f32),
            pltpu.VMEM((CHUNK * LANES,), _f32),
            pltpu.VMEM((CHUNK, M), _f32),
            pltpu.VMEM((ZR, M), _f32),
            pltpu.VMEM_SHARED((NP, M), _f32),
        ],
    )(t_comb, src, dst)


def _sc_aggregate_body(feat_hbm, ee_hbm, rs_hbm, src_hbm, dst_hbm, out_hbm,
                       sidx, didx, rows_v, ee1, rs_v, zrow, acc,
                       *, heads, d16):
    cid = lax.axis_index("c")
    sid = lax.axis_index("s")
    base0 = (cid * NS + sid) * EW

    _zero_shared_slice(zrow, acc, sid)
    plsc.subcore_barrier()

    def chunk_loop(c, _):
        eb = base0 + c * CHUNK
        pltpu.sync_copy(src_hbm.at[pl.ds(eb, CHUNK)], sidx)
        pltpu.sync_copy(dst_hbm.at[pl.ds(eb, CHUNK)], didx)
        pltpu.sync_copy(feat_hbm.at[sidx], rows_v)
        pltpu.sync_copy(rs_hbm.at[didx], rs_v)
        pltpu.sync_copy(ee_hbm.at[pl.ds(eb * LANES, CHUNK * LANES)], ee1)

        def body(e, _):
            grp = ee1[pl.ds(e * LANES, LANES)]
            for h in range(heads):
                sp = jnp.full((LANES,), grp[h], _f32)
                for j in range(d16):
                    sl = pl.ds((h * d16 + j) * LANES, LANES)
                    rows_v[e, sl] = rows_v[e, sl] * sp * rs_v[e, sl]
            return 0

        lax.fori_loop(0, CHUNK, body, 0)
        pltpu.sync_copy(rows_v, acc.at[didx], add=True)
        return 0

    lax.fori_loop(0, NCH, chunk_loop, 0)

    plsc.subcore_barrier()
    row0 = sid * ROWS_T
    pltpu.sync_copy(acc.at[pl.ds(row0, ROWS_T)],
                    out_hbm.at[cid, pl.ds(row0, ROWS_T)])


def _sc_aggregate_stage(feat, ee, rs, src, dst, *, heads, d16):
    mesh = plsc.VectorSubcoreMesh(core_axis_name="c", subcore_axis_name="s",
                                  num_cores=NC, num_subcores=NS)
    return pl.kernel(
        functools.partial(_sc_aggregate_body, heads=heads, d16=d16),
        out_type=jax.ShapeDtypeStruct((NC, NP, M), _f32),
        mesh=mesh,
        scratch_types=[
            pltpu.VMEM((CHUNK,), jnp.int32),
            pltpu.VMEM((CHUNK,), jnp.int32),
            pltpu.VMEM((CHUNK, M), _f32),
            pltpu.VMEM((CHUNK * LANES,), _f32),
            pltpu.VMEM((CHUNK, M), _f32),
            pltpu.VMEM((ZR, M), _f32),
            pltpu.VMEM_SHARED((NP, M), _f32),
        ],
    )(feat, ee, rs, src, dst)


def _pack_att(al, ar):
    h, d = al.shape
    a = jnp.zeros((M, M), _f32)
    rows = jnp.arange(h * d)
    cols = jnp.repeat(jnp.arange(h), d)
    a = a.at[rows, cols].set(al.reshape(-1))
    a = a.at[rows, LANES + cols].set(ar.reshape(-1))
    return a


def _layer(x, src, dst, w, a_comb, *, heads, d16, first):
    feat, t_comb = _dense_stage(x, w, a_comb, first=first)
    ee, s_part = _sc_logits_stage(t_comb, src, dst, heads=heads)
    rs = _recip_stage(s_part)
    return _sc_aggregate_stage(feat, ee, rs, src, dst, heads=heads, d16=d16)


def kernel(inputs, edge_index, W0, al0, ar0, W1, al1, ar1, W2, al2, ar2):
    src = edge_index[0]
    dst = edge_index[1]
    x0 = jnp.pad(inputs, ((0, NP - N), (0, 0)))
    w2p = jnp.pad(W2, ((0, 0), (0, M - W2.shape[1])))
    p = _layer(x0, src, dst, W0, _pack_att(al0, ar0),
               heads=8, d16=1, first=True)
    p = _layer(p, src, dst, W1, _pack_att(al1, ar1),
               heads=8, d16=1, first=False)
    p = _layer(p, src, dst, w2p, _pack_att(al2, ar2),
               heads=1, d16=2, first=False)
    return _combine_stage(p)[:N, :32]


# speedup vs baseline: 27.4136x; 1.1441x over previous
import functools

import jax
import jax.numpy as jnp
from jax import lax
from jax.experimental import pallas as pl
from jax.experimental.pallas import tpu as pltpu
from jax.experimental.pallas import tpu_sc as plsc

N = 10000
NP = 10240
E = 320000
M = 128
LANES = 16
NC = 2
NS = 16
NW = NC * NS
EW = E // NW
CHUNK = 80
NCH = EW // CHUNK
ROWS_T = NP // NS
ZR = 128

_f32 = jnp.float32


def _dense_body(x_ref, s_ref, w_ref, a_ref, feat_ref, t_ref, *, first):
    x = x_ref[...]
    if not first:
        t = (x[0] + x[1]) / (s_ref[0] + s_ref[1] + 1e-9)
        x = jnp.where(t > 0, t, jnp.exp(t) - 1.0)
    feat = jnp.dot(x, w_ref[...], preferred_element_type=_f32)
    feat_ref[...] = feat
    t_ref[...] = jnp.dot(feat, a_ref[...], preferred_element_type=_f32)


def _dense_stage(x, s_part, w, a_comb, *, first):
    bn = 512
    grid = (NP // bn,)
    if first:
        x_spec = pl.BlockSpec((bn, M), lambda i: (i, 0))
    else:
        x_spec = pl.BlockSpec((2, bn, M), lambda i: (0, i, 0))
    return pl.pallas_call(
        functools.partial(_dense_body, first=first),
        grid=grid,
        in_specs=[
            x_spec,
            pl.BlockSpec((2, bn, M), lambda i: (0, i, 0)),
            pl.BlockSpec((M, M), lambda i: (0, 0)),
            pl.BlockSpec((M, M), lambda i: (0, 0)),
        ],
        out_specs=[
            pl.BlockSpec((bn, M), lambda i: (i, 0)),
            pl.BlockSpec((bn, M), lambda i: (i, 0)),
        ],
        out_shape=[
            jax.ShapeDtypeStruct((NP, M), _f32),
            jax.ShapeDtypeStruct((NP, M), _f32),
        ],
    )(x, s_part, w, a_comb)


def _combine_body(p_ref, s_ref, o_ref):
    o_ref[...] = (p_ref[0] + p_ref[1]) / (s_ref[0] + s_ref[1] + 1e-9)


def _combine_stage(p, s_part):
    bn = 512
    return pl.pallas_call(
        _combine_body,
        grid=(NP // bn,),
        in_specs=[
            pl.BlockSpec((2, bn, M), lambda i: (0, i, 0)),
            pl.BlockSpec((2, bn, M), lambda i: (0, i, 0)),
        ],
        out_specs=pl.BlockSpec((bn, M), lambda i: (i, 0)),
        out_shape=jax.ShapeDtypeStruct((NP, M), _f32),
    )(p, s_part)


def _zero_shared_slice(zrow, shared, sid):
    def zfill(i, _):
        r = i // (M // LANES)
        g = i % (M // LANES)
        zrow[r, pl.ds(g * LANES, LANES)] = jnp.zeros((LANES,), _f32)
        return 0

    lax.fori_loop(0, ZR * (M // LANES), zfill, 0)
    row0 = sid * ROWS_T
    for j in range(ROWS_T // ZR):
        pltpu.sync_copy(zrow, shared.at[pl.ds(row0 + j * ZR, ZR)])


def _sc_logits_body(t_hbm, src_hbm, dst_hbm, ee_hbm, spart_hbm,
                    sidx, didx, ts_v, td_v, ee1, eew, zrow, sacc,
                    *, heads):
    cid = lax.axis_index("c")
    sid = lax.axis_index("s")
    base0 = (cid * NS + sid) * EW

    _zero_shared_slice(zrow, sacc, sid)
    plsc.subcore_barrier()

    lane = lax.broadcasted_iota(jnp.int32, (LANES,), 0)

    def chunk_loop(c, _):
        eb = base0 + c * CHUNK
        pltpu.sync_copy(src_hbm.at[pl.ds(eb, CHUNK)], sidx)
        pltpu.sync_copy(dst_hbm.at[pl.ds(eb, CHUNK)], didx)
        pltpu.sync_copy(t_hbm.at[sidx], ts_v)
        pltpu.sync_copy(t_hbm.at[didx], td_v)

        def body(e, _):
            x = ts_v[e, pl.ds(0, LANES)] + td_v[e, pl.ds(LANES, LANES)]
            y = jnp.maximum(x, 0.2 * x)
            z = jnp.where(lane < heads, jnp.exp(y), 0.0)
            ee1[pl.ds(e * LANES, LANES)] = z
            if heads == 8:
                for g in range(8):
                    sp = jnp.full((LANES,), z[g], _f32)
                    eew[e, pl.ds(g * LANES, LANES)] = sp
            else:
                sp = jnp.full((LANES,), z[0], _f32)
                for g in range(8):
                    eew[e, pl.ds(g * LANES, LANES)] = sp
            return 0

        lax.fori_loop(0, CHUNK, body, 0)
        pltpu.sync_copy(ee1, ee_hbm.at[pl.ds(eb * LANES, CHUNK * LANES)])
        pltpu.sync_copy(eew, sacc.at[didx], add=True)
        return 0

    lax.fori_loop(0, NCH, chunk_loop, 0)

    plsc.subcore_barrier()
    row0 = sid * ROWS_T
    pltpu.sync_copy(sacc.at[pl.ds(row0, ROWS_T)],
                    spart_hbm.at[cid, pl.ds(row0, ROWS_T)])


def _sc_logits_stage(t_comb, src, dst, *, heads):
    mesh = plsc.VectorSubcoreMesh(core_axis_name="c", subcore_axis_name="s",
                                  num_cores=NC, num_subcores=NS)
    return pl.kernel(
        functools.partial(_sc_logits_body, heads=heads),
        out_type=(
            jax.ShapeDtypeStruct((E * LANES,), _f32),
            jax.ShapeDtypeStruct((NC, NP, M), _f32),
        ),
        mesh=mesh,
        scratch_types=[
            pltpu.VMEM((CHUNK,), jnp.int32),
            pltpu.VMEM((CHUNK,), jnp.int32),
            pltpu.VMEM((CHUNK, M), _f32),
            pltpu.VMEM((CHUNK, M), _f32),
            pltpu.VMEM((CHUNK * LANES,), _f32),
            pltpu.VMEM((CHUNK, M), _f32),
            pltpu.VMEM((ZR, M), _f32),
            pltpu.VMEM_SHARED((NP, M), _f32),
        ],
    )(t_comb, src, dst)


def _sc_aggregate_body(feat_hbm, ee_hbm, src_hbm, dst_hbm, out_hbm,
                       sidx, didx, rows_v, ee1, zrow, acc,
                       *, heads, d16):
    cid = lax.axis_index("c")
    sid = lax.axis_index("s")
    base0 = (cid * NS + sid) * EW

    _zero_shared_slice(zrow, acc, sid)
    plsc.subcore_barrier()

    def chunk_loop(c, _):
        eb = base0 + c * CHUNK
        pltpu.sync_copy(src_hbm.at[pl.ds(eb, CHUNK)], sidx)
        pltpu.sync_copy(dst_hbm.at[pl.ds(eb, CHUNK)], didx)
        pltpu.sync_copy(feat_hbm.at[sidx], rows_v)
        pltpu.sync_copy(ee_hbm.at[pl.ds(eb * LANES, CHUNK * LANES)], ee1)

        def body(e, _):
            grp = ee1[pl.ds(e * LANES, LANES)]
            for h in range(heads):
                sp = jnp.full((LANES,), grp[h], _f32)
                for j in range(d16):
                    sl = pl.ds((h * d16 + j) * LANES, LANES)
                    rows_v[e, sl] = rows_v[e, sl] * sp
            return 0

        lax.fori_loop(0, CHUNK, body, 0)
        pltpu.sync_copy(rows_v, acc.at[didx], add=True)
        return 0

    lax.fori_loop(0, NCH, chunk_loop, 0)

    plsc.subcore_barrier()
    row0 = sid * ROWS_T
    pltpu.sync_copy(acc.at[pl.ds(row0, ROWS_T)],
                    out_hbm.at[cid, pl.ds(row0, ROWS_T)])


def _sc_aggregate_stage(feat, ee, src, dst, *, heads, d16):
    mesh = plsc.VectorSubcoreMesh(core_axis_name="c", subcore_axis_name="s",
                                  num_cores=NC, num_subcores=NS)
    return pl.kernel(
        functools.partial(_sc_aggregate_body, heads=heads, d16=d16),
        out_type=jax.ShapeDtypeStruct((NC, NP, M), _f32),
        mesh=mesh,
        scratch_types=[
            pltpu.VMEM((CHUNK,), jnp.int32),
            pltpu.VMEM((CHUNK,), jnp.int32),
            pltpu.VMEM((CHUNK, M), _f32),
            pltpu.VMEM((CHUNK * LANES,), _f32),
            pltpu.VMEM((ZR, M), _f32),
            pltpu.VMEM_SHARED((NP, M), _f32),
        ],
    )(feat, ee, src, dst)


def _pack_att(al, ar):
    h, d = al.shape
    a = jnp.zeros((M, M), _f32)
    rows = jnp.arange(h * d)
    cols = jnp.repeat(jnp.arange(h), d)
    a = a.at[rows, cols].set(al.reshape(-1))
    a = a.at[rows, LANES + cols].set(ar.reshape(-1))
    return a


def _layer(x, s_prev, src, dst, w, a_comb, *, heads, d16, first):
    feat, t_comb = _dense_stage(x, s_prev, w, a_comb, first=first)
    ee, s_part = _sc_logits_stage(t_comb, src, dst, heads=heads)
    p = _sc_aggregate_stage(feat, ee, src, dst, heads=heads, d16=d16)
    return p, s_part


def kernel(inputs, edge_index, W0, al0, ar0, W1, al1, ar1, W2, al2, ar2):
    src = edge_index[0]
    dst = edge_index[1]
    x0 = jnp.pad(inputs, ((0, NP - N), (0, 0)))
    w2p = jnp.pad(W2, ((0, 0), (0, M - W2.shape[1])))
    s0 = jnp.zeros((2, NP, M), _f32)
    p, s = _layer(x0, s0, src, dst, W0, _pack_att(al0, ar0),
                  heads=8, d16=1, first=True)
    p, s = _layer(p, s, src, dst, W1, _pack_att(al1, ar1),
                  heads=8, d16=1, first=False)
    p, s = _layer(p, s, src, dst, w2p, _pack_att(al2, ar2),
                  heads=1, d16=2, first=False)
    return _combine_stage(p, s)[:N, :32]


# speedup vs baseline: 31.3656x; 1.1442x over previous
import functools

import jax
import jax.numpy as jnp
from jax import lax
from jax.experimental import pallas as pl
from jax.experimental.pallas import tpu as pltpu
from jax.experimental.pallas import tpu_sc as plsc

N = 10000
NP = 10240
E = 320000
M = 128
LANES = 16
NC = 2
NS = 16
NW = NC * NS
EW = E // NW
CHUNK = 80
NCH = EW // CHUNK
ROWS_T = NP // NS
ZR = 128

_f32 = jnp.float32


def _dense_body(x_ref, s_ref, w_ref, a_ref, feat_ref, t_ref, *, first):
    x = x_ref[...]
    if not first:
        t = (x[0] + x[1]) / (s_ref[0] + s_ref[1] + 1e-9)
        x = jnp.where(t > 0, t, jnp.exp(t) - 1.0)
    feat = jnp.dot(x, w_ref[...], preferred_element_type=_f32)
    feat_ref[...] = feat
    t_ref[...] = jnp.dot(feat, a_ref[...], preferred_element_type=_f32)


def _dense_stage(x, s_part, w, a_comb, *, first):
    bn = 512
    grid = (NP // bn,)
    if first:
        x_spec = pl.BlockSpec((bn, M), lambda i: (i, 0))
    else:
        x_spec = pl.BlockSpec((2, bn, M), lambda i: (0, i, 0))
    return pl.pallas_call(
        functools.partial(_dense_body, first=first),
        grid=grid,
        in_specs=[
            x_spec,
            pl.BlockSpec((2, bn, M), lambda i: (0, i, 0)),
            pl.BlockSpec((M, M), lambda i: (0, 0)),
            pl.BlockSpec((M, M), lambda i: (0, 0)),
        ],
        out_specs=[
            pl.BlockSpec((bn, M), lambda i: (i, 0)),
            pl.BlockSpec((bn, M), lambda i: (i, 0)),
        ],
        out_shape=[
            jax.ShapeDtypeStruct((NP, M), _f32),
            jax.ShapeDtypeStruct((NP, M), _f32),
        ],
    )(x, s_part, w, a_comb)


def _combine_body(p_ref, s_ref, o_ref):
    o_ref[...] = (p_ref[0] + p_ref[1]) / (s_ref[0] + s_ref[1] + 1e-9)


def _combine_stage(p, s_part):
    bn = 512
    return pl.pallas_call(
        _combine_body,
        grid=(NP // bn,),
        in_specs=[
            pl.BlockSpec((2, bn, M), lambda i: (0, i, 0)),
            pl.BlockSpec((2, bn, M), lambda i: (0, i, 0)),
        ],
        out_specs=pl.BlockSpec((bn, M), lambda i: (i, 0)),
        out_shape=jax.ShapeDtypeStruct((NP, M), _f32),
    )(p, s_part)


def _zero_shared_slice(zrow, shared, sid):
    @plsc.parallel_loop(0, ZR * (M // LANES), unroll=8)
    def _(i):
        r = i // (M // LANES)
        g = i % (M // LANES)
        zrow[r, pl.ds(g * LANES, LANES)] = jnp.zeros((LANES,), _f32)
    row0 = sid * ROWS_T
    for j in range(ROWS_T // ZR):
        pltpu.sync_copy(zrow, shared.at[pl.ds(row0 + j * ZR, ZR)])


def _sc_logits_body(t_hbm, src_hbm, dst_hbm, ee_hbm, spart_hbm,
                    sidx, didx, ts_v, td_v, ee1, eew, zrow, sacc,
                    *, heads):
    cid = lax.axis_index("c")
    sid = lax.axis_index("s")
    base0 = (cid * NS + sid) * EW

    _zero_shared_slice(zrow, sacc, sid)
    plsc.subcore_barrier()

    lane = lax.broadcasted_iota(jnp.int32, (LANES,), 0)

    def chunk_loop(c, _):
        eb = base0 + c * CHUNK
        pltpu.sync_copy(src_hbm.at[pl.ds(eb, CHUNK)], sidx)
        pltpu.sync_copy(dst_hbm.at[pl.ds(eb, CHUNK)], didx)
        pltpu.sync_copy(t_hbm.at[sidx], ts_v)
        pltpu.sync_copy(t_hbm.at[didx], td_v)

        @plsc.parallel_loop(0, CHUNK, unroll=4)
        def _(e):
            x = ts_v[e, pl.ds(0, LANES)] + td_v[e, pl.ds(LANES, LANES)]
            y = jnp.maximum(x, 0.2 * x)
            z = jnp.where(lane < heads, jnp.exp(y), 0.0)
            ee1[pl.ds(e * LANES, LANES)] = z
            if heads == 8:
                for g in range(8):
                    sp = jnp.full((LANES,), z[g], _f32)
                    eew[e, pl.ds(g * LANES, LANES)] = sp
            else:
                sp = jnp.full((LANES,), z[0], _f32)
                for g in range(8):
                    eew[e, pl.ds(g * LANES, LANES)] = sp
        pltpu.sync_copy(ee1, ee_hbm.at[pl.ds(eb * LANES, CHUNK * LANES)])
        pltpu.sync_copy(eew, sacc.at[didx], add=True)
        return 0

    lax.fori_loop(0, NCH, chunk_loop, 0)

    plsc.subcore_barrier()
    row0 = sid * ROWS_T
    pltpu.sync_copy(sacc.at[pl.ds(row0, ROWS_T)],
                    spart_hbm.at[cid, pl.ds(row0, ROWS_T)])


def _sc_logits_stage(t_comb, src, dst, *, heads):
    mesh = plsc.VectorSubcoreMesh(core_axis_name="c", subcore_axis_name="s",
                                  num_cores=NC, num_subcores=NS)
    return pl.kernel(
        functools.partial(_sc_logits_body, heads=heads),
        out_type=(
            jax.ShapeDtypeStruct((E * LANES,), _f32),
            jax.ShapeDtypeStruct((NC, NP, M), _f32),
        ),
        mesh=mesh,
        scratch_types=[
            pltpu.VMEM((CHUNK,), jnp.int32),
            pltpu.VMEM((CHUNK,), jnp.int32),
            pltpu.VMEM((CHUNK, M), _f32),
            pltpu.VMEM((CHUNK, M), _f32),
            pltpu.VMEM((CHUNK * LANES,), _f32),
            pltpu.VMEM((CHUNK, M), _f32),
            pltpu.VMEM((ZR, M), _f32),
            pltpu.VMEM_SHARED((NP, M), _f32),
        ],
    )(t_comb, src, dst)


def _sc_aggregate_body(feat_hbm, ee_hbm, src_hbm, dst_hbm, out_hbm,
                       sidx, didx, rows_v, ee1, zrow, acc,
                       *, heads, d16):
    cid = lax.axis_index("c")
    sid = lax.axis_index("s")
    base0 = (cid * NS + sid) * EW

    _zero_shared_slice(zrow, acc, sid)
    plsc.subcore_barrier()

    def chunk_loop(c, _):
        eb = base0 + c * CHUNK
        pltpu.sync_copy(src_hbm.at[pl.ds(eb, CHUNK)], sidx)
        pltpu.sync_copy(dst_hbm.at[pl.ds(eb, CHUNK)], didx)
        pltpu.sync_copy(feat_hbm.at[sidx], rows_v)
        pltpu.sync_copy(ee_hbm.at[pl.ds(eb * LANES, CHUNK * LANES)], ee1)

        @plsc.parallel_loop(0, CHUNK, unroll=4)
        def _(e):
            grp = ee1[pl.ds(e * LANES, LANES)]
            for h in range(heads):
                sp = jnp.full((LANES,), grp[h], _f32)
                for j in range(d16):
                    sl = pl.ds((h * d16 + j) * LANES, LANES)
                    rows_v[e, sl] = rows_v[e, sl] * sp
        pltpu.sync_copy(rows_v, acc.at[didx], add=True)
        return 0

    lax.fori_loop(0, NCH, chunk_loop, 0)

    plsc.subcore_barrier()
    row0 = sid * ROWS_T
    pltpu.sync_copy(acc.at[pl.ds(row0, ROWS_T)],
                    out_hbm.at[cid, pl.ds(row0, ROWS_T)])


def _sc_aggregate_stage(feat, ee, src, dst, *, heads, d16):
    mesh = plsc.VectorSubcoreMesh(core_axis_name="c", subcore_axis_name="s",
                                  num_cores=NC, num_subcores=NS)
    return pl.kernel(
        functools.partial(_sc_aggregate_body, heads=heads, d16=d16),
        out_type=jax.ShapeDtypeStruct((NC, NP, M), _f32),
        mesh=mesh,
        scratch_types=[
            pltpu.VMEM((CHUNK,), jnp.int32),
            pltpu.VMEM((CHUNK,), jnp.int32),
            pltpu.VMEM((CHUNK, M), _f32),
            pltpu.VMEM((CHUNK * LANES,), _f32),
            pltpu.VMEM((ZR, M), _f32),
            pltpu.VMEM_SHARED((NP, M), _f32),
        ],
    )(feat, ee, src, dst)


def _pack_att(al, ar):
    h, d = al.shape
    a = jnp.zeros((M, M), _f32)
    rows = jnp.arange(h * d)
    cols = jnp.repeat(jnp.arange(h), d)
    a = a.at[rows, cols].set(al.reshape(-1))
    a = a.at[rows, LANES + cols].set(ar.reshape(-1))
    return a


def _layer(x, s_prev, src, dst, w, a_comb, *, heads, d16, first):
    feat, t_comb = _dense_stage(x, s_prev, w, a_comb, first=first)
    ee, s_part = _sc_logits_stage(t_comb, src, dst, heads=heads)
    p = _sc_aggregate_stage(feat, ee, src, dst, heads=heads, d16=d16)
    return p, s_part


def kernel(inputs, edge_index, W0, al0, ar0, W1, al1, ar1, W2, al2, ar2):
    src = edge_index[0]
    dst = edge_index[1]
    x0 = jnp.pad(inputs, ((0, NP - N), (0, 0)))
    w2p = jnp.pad(W2, ((0, 0), (0, M - W2.shape[1])))
    s0 = jnp.zeros((2, NP, M), _f32)
    p, s = _layer(x0, s0, src, dst, W0, _pack_att(al0, ar0),
                  heads=8, d16=1, first=True)
    p, s = _layer(p, s, src, dst, W1, _pack_att(al1, ar1),
                  heads=8, d16=1, first=False)
    p, s = _layer(p, s, src, dst, w2p, _pack_att(al2, ar2),
                  heads=1, d16=2, first=False)
    return _combine_stage(p, s)[:N, :32]
